# baseline (speedup 1.0000x reference)
import jax
import jax.numpy as jnp
import numpy as np
from jax import lax
from jax.experimental import pallas as pl
from jax.experimental.pallas import tpu as pltpu
from jax.experimental.pallas import tpu_sc as plsc

N = 10000
E = 160000
D = 128
DA = 16
DE = 4
DEMB = 16
DQK = 64
H = 8

NC, NS = 2, 16
NW = NC * NS
NSLAB = 5
SLAB = E // NSLAB
CH = 128
SPAIR = SLAB // (2 * CH)
BASE_PAIRS = SPAIR // NW
EXTRA_W = SPAIR - BASE_PAIRS * NW
CHS = 80
SPAIR_S = SLAB // (2 * CHS)
BASE_PAIRS_S = SPAIR_S // NW
EXTRA_W_S = SPAIR_S - BASE_PAIRS_S * NW
NP = 10240
ROWS_PER_TILE = NP // NS
NAZ = NP // 128
GCOL = 64

BN = 1000
BE = 4000
BNC = 1024
BZ = BNC // 128

_mesh = lambda: plsc.VectorSubcoreMesh(core_axis_name="c", subcore_axis_name="s")


def _body_a(nf_ref, na_ref, wq_ref, wdot_ref, wsct_ref, qd_ref, sc_ref):
    nf = nf_ref[...]
    rowi = lax.broadcasted_iota(jnp.int32, (BN, 1), 0)
    gtag = ((rowi + pl.program_id(0) * BN) & 127).astype(jnp.float32)
    lane = lax.broadcasted_iota(jnp.int32, (BN, D), 1)
    qd = jnp.dot(jnp.dot(nf, wq_ref[...]), wdot_ref[...])
    qd_ref[...] = qd + jnp.where(lane == GCOL, gtag, 0.0)
    na = na_ref[...]
    acc = jnp.zeros((BN, D), jnp.float32)
    for v in range(DA):
        acc = acc + na[:, v:v + 1] * jnp.dot(nf, wsct_ref[v])
    sc_ref[...] = acc


def _run_a(nf, na, Wq, Wdot_pad, Wsc_t):
    return pl.pallas_call(
        _body_a,
        grid=(N // BN,),
        in_specs=[
            pl.BlockSpec((BN, D), lambda i: (i, 0)),
            pl.BlockSpec((BN, DA), lambda i: (i, 0)),
            pl.BlockSpec((D, DQK), lambda i: (0, 0)),
            pl.BlockSpec((DQK, D), lambda i: (0, 0)),
            pl.BlockSpec((DA, D, D), lambda i: (0, 0, 0)),
        ],
        out_specs=[
            pl.BlockSpec((BN, D), lambda i: (i, 0)),
            pl.BlockSpec((BN, D), lambda i: (i, 0)),
        ],
        out_shape=[
            jax.ShapeDtypeStruct((N, D), jnp.float32),
            jax.ShapeDtypeStruct((N, D), jnp.float32),
        ],
    )(nf, na, Wq, Wdot_pad, Wsc_t)


def _gather_body(nf_hbm, qd_hbm, src_hbm, dst_hbm, xs_hbm, qdd_hbm,
                 sidxa, didxa, sidxb, didxb, xra, qra, xrb, qrb,
                 gxa, gqa, gxb, gqb, sxa, sqa, sxb, sqb):
    c = lax.axis_index("c")
    s = lax.axis_index("s")
    w = s * NC + c
    npr = BASE_PAIRS + jnp.where(w < EXTRA_W, 1, 0)

    def body(i, carry):
        base = (w + i * NW) * 2 * CH

        @pl.when(i > 0)
        def _():
            pltpu.make_async_copy(xra, xs_hbm.at[pl.ds(base, CH)], sxa).wait()
            pltpu.make_async_copy(qra, qdd_hbm.at[pl.ds(base, CH)], sqa).wait()
            pltpu.make_async_copy(xrb, xs_hbm.at[pl.ds(base, CH)], sxb).wait()
            pltpu.make_async_copy(qrb, qdd_hbm.at[pl.ds(base, CH)], sqb).wait()

        pltpu.sync_copy(src_hbm.at[pl.ds(base, CH)], sidxa)
        pltpu.sync_copy(dst_hbm.at[pl.ds(base, CH)], didxa)
        pltpu.sync_copy(src_hbm.at[pl.ds(base + CH, CH)], sidxb)
        pltpu.sync_copy(dst_hbm.at[pl.ds(base + CH, CH)], didxb)
        ca1 = pltpu.async_copy(nf_hbm.at[sidxa], xra, gxa)
        ca2 = pltpu.async_copy(qd_hbm.at[didxa], qra, gqa)
        cb1 = pltpu.async_copy(nf_hbm.at[sidxb], xrb, gxb)
        cb2 = pltpu.async_copy(qd_hbm.at[didxb], qrb, gqb)
        ca1.wait()
        ca2.wait()
        pltpu.async_copy(xra, xs_hbm.at[pl.ds(base, CH)], sxa)
        pltpu.async_copy(qra, qdd_hbm.at[pl.ds(base, CH)], sqa)
        cb1.wait()
        cb2.wait()
        pltpu.async_copy(xrb, xs_hbm.at[pl.ds(base + CH, CH)], sxb)
        pltpu.async_copy(qrb, qdd_hbm.at[pl.ds(base + CH, CH)], sqb)
        return carry

    lax.fori_loop(0, npr, body, 0)
    pltpu.make_async_copy(xra, xs_hbm.at[pl.ds(0, CH)], sxa).wait()
    pltpu.make_async_copy(qra, qdd_hbm.at[pl.ds(0, CH)], sqa).wait()
    pltpu.make_async_copy(xrb, xs_hbm.at[pl.ds(0, CH)], sxb).wait()
    pltpu.make_async_copy(qrb, qdd_hbm.at[pl.ds(0, CH)], sqb).wait()


def _run_gather(nf, qd, src_s, dst_s):
    fn = pl.kernel(
        _gather_body,
        out_type=(
            jax.ShapeDtypeStruct((SLAB, D), jnp.float32),
            jax.ShapeDtypeStruct((SLAB, D), jnp.float32),
        ),
        mesh=_mesh(),
        scratch_types=[
            pltpu.VMEM((CH,), jnp.int32),
            pltpu.VMEM((CH,), jnp.int32),
            pltpu.VMEM((CH,), jnp.int32),
            pltpu.VMEM((CH,), jnp.int32),
            pltpu.VMEM((CH, D), jnp.float32),
            pltpu.VMEM((CH, D), jnp.float32),
            pltpu.VMEM((CH, D), jnp.float32),
            pltpu.VMEM((CH, D), jnp.float32),
        ] + [pltpu.SemaphoreType.DMA] * 8,
    )
    return fn(nf, qd, src_s, dst_s)


def _body_b(xs_ref, qdd_ref, ee_ref, ea_ref, w1k_ref, w2kr_ref, wk_ref,
            w1v_ref, w2vr_ref, wv_ref, r_ref, t_ref, selg_ref,
            m_ref, ze_ref):
    xs = xs_ref[...]
    qdd = qdd_ref[...]
    ee = ee_ref[...]
    ea2 = jnp.dot(ea_ref[...], t_ref[...])
    r = r_ref[...]

    hk = jnp.dot(ee, w1k_ref[...])
    hk = hk * jax.nn.sigmoid(hk)
    ak = jnp.dot(jnp.dot(hk, r) * ea2, w2kr_ref[...])
    k = jnp.dot(xs * ak, wk_ref[...])
    dot = jnp.sum(qdd * k, axis=1, keepdims=True)
    se = jnp.exp(0.5 * dot - 0.05)

    hv = jnp.dot(ee, w1v_ref[...])
    hv = hv * jax.nn.sigmoid(hv)
    av = jnp.dot(jnp.dot(hv, r) * ea2, w2vr_ref[...])
    v = jnp.dot(xs * av, wv_ref[...])

    m_ref[...] = se * v
    g = jnp.dot(qdd, selg_ref[...]).astype(jnp.int32)
    lane = lax.broadcasted_iota(jnp.int32, (BE, D), 1)
    ze_ref[...] = jnp.where(lane == g, se * se, 0.0)


def _run_b(xs, qdd, ee, ea, W1k, W2k_r, Wk_pad, W1v, W2v_r, Wv, R, T, selg,
           blk_off):
    return pl.pallas_call(
        _body_b,
        grid=(SLAB // BE,),
        in_specs=[
            pl.BlockSpec((BE, D), lambda i: (i, 0)),
            pl.BlockSpec((BE, D), lambda i: (i, 0)),
            pl.BlockSpec((BE, DEMB), lambda i: (i + blk_off, 0)),
            pl.BlockSpec((BE, DE), lambda i: (i + blk_off, 0)),
            pl.BlockSpec((DEMB, H), lambda i: (0, 0)),
            pl.BlockSpec((H * DE, D), lambda i: (0, 0)),
            pl.BlockSpec((D, D), lambda i: (0, 0)),
            pl.BlockSpec((DEMB, H), lambda i: (0, 0)),
            pl.BlockSpec((H * DE, D), lambda i: (0, 0)),
            pl.BlockSpec((D, D), lambda i: (0, 0)),
            pl.BlockSpec((H, H * DE), lambda i: (0, 0)),
            pl.BlockSpec((DE, H * DE), lambda i: (0, 0)),
            pl.BlockSpec((D, 1), lambda i: (0, 0)),
        ],
        out_specs=[
            pl.BlockSpec((BE, D), lambda i: (i, 0)),
            pl.BlockSpec((BE, D), lambda i: (i, 0)),
        ],
        out_shape=[
            jax.ShapeDtypeStruct((SLAB, D), jnp.float32),
            jax.ShapeDtypeStruct((SLAB, D), jnp.float32),
        ],
    )(xs, qdd, ee, ea, W1k, W2k_r, Wk_pad, W1v, W2v_r, Wv, R, T, selg)


def _make_scatter_body(nslabs):
    def body_fn(*refs):
        slab_refs = [tuple(refs[3 * t:3 * t + 3]) for t in range(nslabs)]
        zer_hbm, am_hbm, az_hbm = refs[3 * nslabs:3 * nslabs + 3]
        (didxa, didx8a, didxb, didx8b, mra, zra, mrb, zrb,
         accm, accz, lma, lza, lmb, lzb) = refs[3 * nslabs + 3:]
        c = lax.axis_index("c")
        s = lax.axis_index("s")
        w = s * NC + c
        npr = BASE_PAIRS_S + jnp.where(w < EXTRA_W_S, 1, 0)

        pltpu.sync_copy(zer_hbm,
                        accm.at[pl.ds(s * ROWS_PER_TILE, ROWS_PER_TILE)])

        @pl.when(s < NAZ // 8)
        def _():
            pltpu.sync_copy(zer_hbm.at[pl.ds(0, 8)], accz.at[pl.ds(s * 8, 8)])

        plsc.subcore_barrier()

        for m_hbm, ze_hbm, dst_hbm in slab_refs:
            def body(i, carry):
                base = (w + i * NW) * 2 * CHS
                cma = pltpu.async_copy(m_hbm.at[pl.ds(base, CHS)], mra, lma)
                cza = pltpu.async_copy(ze_hbm.at[pl.ds(base, CHS)], zra, lza)
                cmb = pltpu.async_copy(
                    m_hbm.at[pl.ds(base + CHS, CHS)], mrb, lmb)
                czb = pltpu.async_copy(
                    ze_hbm.at[pl.ds(base + CHS, CHS)], zrb, lzb)
                pltpu.sync_copy(dst_hbm.at[pl.ds(base, CHS)], didxa)
                pltpu.sync_copy(dst_hbm.at[pl.ds(base + CHS, CHS)], didxb)
                for j in range(CHS // 16):
                    didx8a[pl.ds(j * 16, 16)] = lax.shift_right_logical(
                        didxa[pl.ds(j * 16, 16)], 7)
                    didx8b[pl.ds(j * 16, 16)] = lax.shift_right_logical(
                        didxb[pl.ds(j * 16, 16)], 7)
                cma.wait()
                cza.wait()
                pltpu.sync_copy(mra, accm.at[didxa], add=True)
                pltpu.sync_copy(zra, accz.at[didx8a], add=True)
                cmb.wait()
                czb.wait()
                pltpu.sync_copy(mrb, accm.at[didxb], add=True)
                pltpu.sync_copy(zrb, accz.at[didx8b], add=True)
                return carry

            lax.fori_loop(0, npr, body, 0)

        plsc.subcore_barrier()
        pltpu.sync_copy(
            accm.at[pl.ds(s * ROWS_PER_TILE, ROWS_PER_TILE)],
            am_hbm.at[c, pl.ds(s * ROWS_PER_TILE, ROWS_PER_TILE)])

        @pl.when(s < NAZ // 8)
        def _():
            pltpu.sync_copy(accz.at[pl.ds(s * 8, 8)],
                            az_hbm.at[c, pl.ds(s * 8, 8)])

    return body_fn


def _run_scatter(slabs, zer):
    fn = pl.kernel(
        _make_scatter_body(len(slabs)),
        out_type=(
            jax.ShapeDtypeStruct((NC, NP, D), jnp.float32),
            jax.ShapeDtypeStruct((NC, NAZ, D), jnp.float32),
        ),
        mesh=_mesh(),
        scratch_types=[
            pltpu.VMEM((CHS,), jnp.int32),
            pltpu.VMEM((CHS,), jnp.int32),
            pltpu.VMEM((CHS,), jnp.int32),
            pltpu.VMEM((CHS,), jnp.int32),
            pltpu.VMEM((CHS, D), jnp.float32),
            pltpu.VMEM((CHS, D), jnp.float32),
            pltpu.VMEM((CHS, D), jnp.float32),
            pltpu.VMEM((CHS, D), jnp.float32),
            pltpu.VMEM_SHARED((NP, D), jnp.float32),
            pltpu.VMEM_SHARED((NAZ, D), jnp.float32),
        ] + [pltpu.SemaphoreType.DMA] * 4,
    )
    args = []
    for t in slabs:
        args.extend(t)
    return fn(*args, zer)


def _body_c(am1_ref, az1_ref, am2_ref, az2_ref, sc_ref, rsel_ref, out_ref):
    stot = am1_ref[0] + am1_ref[1] + am2_ref[0] + am2_ref[1]
    azs = az1_ref[0] + az1_ref[1] + az2_ref[0] + az2_ref[1]
    b1 = jnp.dot(rsel_ref[...], azs)
    rowi = lax.broadcasted_iota(jnp.int32, (BNC, 1), 0)
    lane = lax.broadcasted_iota(jnp.int32, (BNC, D), 1)
    zmask = lane == (rowi & 127)
    z = jnp.sum(jnp.where(zmask, b1, 0.0), axis=1, keepdims=True)
    z = jnp.where(z == 0.0, 1.0, z)
    out_ref[...] = stot * lax.rsqrt(z) + sc_ref[...]


def _run_c(am1, az1, am2, az2, sc_pad, rsel):
    return pl.pallas_call(
        _body_c,
        grid=(NP // BNC,),
        in_specs=[
            pl.BlockSpec((NC, BNC, D), lambda i: (0, i, 0)),
            pl.BlockSpec((NC, BZ, D), lambda i: (0, i, 0)),
            pl.BlockSpec((NC, BNC, D), lambda i: (0, i, 0)),
            pl.BlockSpec((NC, BZ, D), lambda i: (0, i, 0)),
            pl.BlockSpec((BNC, D), lambda i: (i, 0)),
            pl.BlockSpec((BNC, BZ), lambda i: (0, 0)),
        ],
        out_specs=pl.BlockSpec((BNC, D), lambda i: (i, 0)),
        out_shape=jax.ShapeDtypeStruct((NP, D), jnp.float32),
    )(am1, az1, am2, az2, sc_pad, rsel)


_R_EXPAND = np.repeat(np.eye(H, dtype=np.float32), DE, axis=1)
_T_EXPAND = np.tile(np.eye(DE, dtype=np.float32), (1, H))
_SELG = np.zeros((D, 1), dtype=np.float32)
_SELG[GCOL, 0] = 1.0
_RSEL = np.zeros((BNC, BZ), dtype=np.float32)
_RSEL[np.arange(BNC), np.arange(BNC) // 128] = 1.0


@jax.jit
def kernel(node_features, node_attrs, edge_embedding, edge_attrs, edge_index,
           positions, Wq, W1k, W2k, Wk, W1v, W2v, Wv, Wdot, Wsc):
    src = edge_index[0].astype(jnp.int32)
    dst = edge_index[1].astype(jnp.int32)

    Wsc_t = jnp.transpose(Wsc, (1, 0, 2))
    W2k_r = jnp.transpose(W2k, (0, 2, 1)).reshape(H * DE, D)
    W2v_r = jnp.transpose(W2v, (0, 2, 1)).reshape(H * DE, D)
    Wdot_pad = jnp.concatenate(
        [Wdot, jnp.zeros((DQK, D - DQK), jnp.float32)], axis=1)
    Wk_pad = jnp.concatenate(
        [Wk, jnp.zeros((D, D - DQK), jnp.float32)], axis=1)
    R = jnp.asarray(_R_EXPAND)
    T = jnp.asarray(_T_EXPAND)
    selg = jnp.asarray(_SELG)
    rsel = jnp.asarray(_RSEL)
    zer = jnp.zeros((ROWS_PER_TILE, D), jnp.float32)

    qd, sc = _run_a(node_features, node_attrs, Wq, Wdot_pad, Wsc_t)

    ms, zes, ds = [], [], []
    for sl in range(NSLAB):
        lo, hi = sl * SLAB, (sl + 1) * SLAB
        d_s = dst[lo:hi]
        xs, qdd = _run_gather(node_features, qd, src[lo:hi], d_s)
        m, ze = _run_b(xs, qdd, edge_embedding, edge_attrs,
                       W1k, W2k_r, Wk_pad, W1v, W2v_r, Wv, R, T, selg,
                       sl * (SLAB // BE))
        ms.append(m)
        zes.append(ze)
        ds.append(d_s)

    am1, az1 = _run_scatter(
        [(ms[0], zes[0], ds[0]), (ms[1], zes[1], ds[1])], zer)
    am2, az2 = _run_scatter(
        [(ms[2], zes[2], ds[2]), (ms[3], zes[3], ds[3]),
         (ms[4], zes[4], ds[4])], zer)
    sc_pad = jnp.pad(sc, ((0, NP - N), (0, 0)))
    return _run_c(am1, az1, am2, az2, sc_pad, rsel)[:N]


# speedup vs baseline: 1.0453x; 1.0453x over previous
import jax
import jax.numpy as jnp
import numpy as np
from jax import lax
from jax.experimental import pallas as pl
from jax.experimental.pallas import tpu as pltpu
from jax.experimental.pallas import tpu_sc as plsc

N = 10000
E = 160000
D = 128
DA = 16
DE = 4
DEMB = 16
DQK = 64
H = 8

NC, NS = 2, 16
NW = NC * NS
NSLAB = 5
SLAB = E // NSLAB
CH = 128
SPAIR = SLAB // (2 * CH)
BASE_PAIRS = SPAIR // NW
EXTRA_W = SPAIR - BASE_PAIRS * NW
CHS = 80
SPAIR_S = SLAB // (2 * CHS)
BASE_PAIRS_S = SPAIR_S // NW
EXTRA_W_S = SPAIR_S - BASE_PAIRS_S * NW
NP = 10240
ROWS_PER_TILE = NP // NS
NAZ = NP // 128
GCOL = 64

BN = 1000
BE = 4000
BNC = 1024
BZ = BNC // 128

_mesh = lambda: plsc.VectorSubcoreMesh(core_axis_name="c", subcore_axis_name="s")


def _body_a1(nf_ref, wq_ref, wdot_ref, qd_ref):
    nf = nf_ref[...]
    rowi = lax.broadcasted_iota(jnp.int32, (BN, 1), 0)
    gtag = ((rowi + pl.program_id(0) * BN) & 127).astype(jnp.float32)
    lane = lax.broadcasted_iota(jnp.int32, (BN, D), 1)
    qd = jnp.dot(jnp.dot(nf, wq_ref[...]), wdot_ref[...])
    qd_ref[...] = qd + jnp.where(lane == GCOL, gtag, 0.0)


def _run_a1(nf, Wq, Wdot_pad):
    return pl.pallas_call(
        _body_a1,
        grid=(N // BN,),
        in_specs=[
            pl.BlockSpec((BN, D), lambda i: (i, 0)),
            pl.BlockSpec((D, DQK), lambda i: (0, 0)),
            pl.BlockSpec((DQK, D), lambda i: (0, 0)),
        ],
        out_specs=pl.BlockSpec((BN, D), lambda i: (i, 0)),
        out_shape=jax.ShapeDtypeStruct((N, D), jnp.float32),
    )(nf, Wq, Wdot_pad)


def _body_a2(nf_ref, na_ref, wsct_ref, sc_ref):
    nf = nf_ref[...]
    na = na_ref[...]
    acc = jnp.zeros((BN, D), jnp.float32)
    for v in range(DA):
        acc = acc + na[:, v:v + 1] * jnp.dot(nf, wsct_ref[v])
    sc_ref[...] = acc


def _run_a2(nf, na, Wsc_t):
    return pl.pallas_call(
        _body_a2,
        grid=(N // BN,),
        in_specs=[
            pl.BlockSpec((BN, D), lambda i: (i, 0)),
            pl.BlockSpec((BN, DA), lambda i: (i, 0)),
            pl.BlockSpec((DA, D, D), lambda i: (0, 0, 0)),
        ],
        out_specs=pl.BlockSpec((BN, D), lambda i: (i, 0)),
        out_shape=jax.ShapeDtypeStruct((N, D), jnp.float32),
    )(nf, na, Wsc_t)


def _gather_body(nf_hbm, qd_hbm, src_hbm, dst_hbm, xs_hbm, qdd_hbm,
                 sidxa, didxa, sidxb, didxb, xra, qra, xrb, qrb,
                 gxa, gqa, gxb, gqb, sxa, sqa, sxb, sqb):
    c = lax.axis_index("c")
    s = lax.axis_index("s")
    w = s * NC + c
    npr = BASE_PAIRS + jnp.where(w < EXTRA_W, 1, 0)

    def body(i, carry):
        base = (w + i * NW) * 2 * CH

        @pl.when(i > 0)
        def _():
            pltpu.make_async_copy(xra, xs_hbm.at[pl.ds(base, CH)], sxa).wait()
            pltpu.make_async_copy(qra, qdd_hbm.at[pl.ds(base, CH)], sqa).wait()
            pltpu.make_async_copy(xrb, xs_hbm.at[pl.ds(base, CH)], sxb).wait()
            pltpu.make_async_copy(qrb, qdd_hbm.at[pl.ds(base, CH)], sqb).wait()

        pltpu.sync_copy(src_hbm.at[pl.ds(base, CH)], sidxa)
        pltpu.sync_copy(dst_hbm.at[pl.ds(base, CH)], didxa)
        pltpu.sync_copy(src_hbm.at[pl.ds(base + CH, CH)], sidxb)
        pltpu.sync_copy(dst_hbm.at[pl.ds(base + CH, CH)], didxb)
        ca1 = pltpu.async_copy(nf_hbm.at[sidxa], xra, gxa)
        ca2 = pltpu.async_copy(qd_hbm.at[didxa], qra, gqa)
        cb1 = pltpu.async_copy(nf_hbm.at[sidxb], xrb, gxb)
        cb2 = pltpu.async_copy(qd_hbm.at[didxb], qrb, gqb)
        ca1.wait()
        ca2.wait()
        pltpu.async_copy(xra, xs_hbm.at[pl.ds(base, CH)], sxa)
        pltpu.async_copy(qra, qdd_hbm.at[pl.ds(base, CH)], sqa)
        cb1.wait()
        cb2.wait()
        pltpu.async_copy(xrb, xs_hbm.at[pl.ds(base + CH, CH)], sxb)
        pltpu.async_copy(qrb, qdd_hbm.at[pl.ds(base + CH, CH)], sqb)
        return carry

    lax.fori_loop(0, npr, body, 0)
    pltpu.make_async_copy(xra, xs_hbm.at[pl.ds(0, CH)], sxa).wait()
    pltpu.make_async_copy(qra, qdd_hbm.at[pl.ds(0, CH)], sqa).wait()
    pltpu.make_async_copy(xrb, xs_hbm.at[pl.ds(0, CH)], sxb).wait()
    pltpu.make_async_copy(qrb, qdd_hbm.at[pl.ds(0, CH)], sqb).wait()


def _run_gather(nf, qd, src_s, dst_s):
    fn = pl.kernel(
        _gather_body,
        out_type=(
            jax.ShapeDtypeStruct((SLAB, D), jnp.float32),
            jax.ShapeDtypeStruct((SLAB, D), jnp.float32),
        ),
        mesh=_mesh(),
        scratch_types=[
            pltpu.VMEM((CH,), jnp.int32),
            pltpu.VMEM((CH,), jnp.int32),
            pltpu.VMEM((CH,), jnp.int32),
            pltpu.VMEM((CH,), jnp.int32),
            pltpu.VMEM((CH, D), jnp.float32),
            pltpu.VMEM((CH, D), jnp.float32),
            pltpu.VMEM((CH, D), jnp.float32),
            pltpu.VMEM((CH, D), jnp.float32),
        ] + [pltpu.SemaphoreType.DMA] * 8,
    )
    return fn(nf, qd, src_s, dst_s)


def _body_b(xs_ref, qdd_ref, ee_ref, ea_ref, w1k_ref, w2kr_ref, wk_ref,
            w1v_ref, w2vr_ref, wv_ref, r_ref, t_ref, selg_ref,
            m_ref, ze_ref):
    xs = xs_ref[...]
    qdd = qdd_ref[...]
    ee = ee_ref[...]
    ea2 = jnp.dot(ea_ref[...], t_ref[...])
    r = r_ref[...]

    hk = jnp.dot(ee, w1k_ref[...])
    hk = hk * jax.nn.sigmoid(hk)
    ak = jnp.dot(jnp.dot(hk, r) * ea2, w2kr_ref[...])
    k = jnp.dot(xs * ak, wk_ref[...])
    dot = jnp.sum(qdd * k, axis=1, keepdims=True)
    se = jnp.exp(0.5 * dot - 0.05)

    hv = jnp.dot(ee, w1v_ref[...])
    hv = hv * jax.nn.sigmoid(hv)
    av = jnp.dot(jnp.dot(hv, r) * ea2, w2vr_ref[...])
    v = jnp.dot(xs * av, wv_ref[...])

    m_ref[...] = se * v
    g = jnp.dot(qdd, selg_ref[...]).astype(jnp.int32)
    lane = lax.broadcasted_iota(jnp.int32, (BE, D), 1)
    ze_ref[...] = jnp.where(lane == g, se * se, 0.0)


def _run_b(xs, qdd, ee, ea, W1k, W2k_r, Wk_pad, W1v, W2v_r, Wv, R, T, selg,
           blk_off):
    return pl.pallas_call(
        _body_b,
        grid=(SLAB // BE,),
        in_specs=[
            pl.BlockSpec((BE, D), lambda i: (i, 0)),
            pl.BlockSpec((BE, D), lambda i: (i, 0)),
            pl.BlockSpec((BE, DEMB), lambda i: (i + blk_off, 0)),
            pl.BlockSpec((BE, DE), lambda i: (i + blk_off, 0)),
            pl.BlockSpec((DEMB, H), lambda i: (0, 0)),
            pl.BlockSpec((H * DE, D), lambda i: (0, 0)),
            pl.BlockSpec((D, D), lambda i: (0, 0)),
            pl.BlockSpec((DEMB, H), lambda i: (0, 0)),
            pl.BlockSpec((H * DE, D), lambda i: (0, 0)),
            pl.BlockSpec((D, D), lambda i: (0, 0)),
            pl.BlockSpec((H, H * DE), lambda i: (0, 0)),
            pl.BlockSpec((DE, H * DE), lambda i: (0, 0)),
            pl.BlockSpec((D, 1), lambda i: (0, 0)),
        ],
        out_specs=[
            pl.BlockSpec((BE, D), lambda i: (i, 0)),
            pl.BlockSpec((BE, D), lambda i: (i, 0)),
        ],
        out_shape=[
            jax.ShapeDtypeStruct((SLAB, D), jnp.float32),
            jax.ShapeDtypeStruct((SLAB, D), jnp.float32),
        ],
    )(xs, qdd, ee, ea, W1k, W2k_r, Wk_pad, W1v, W2v_r, Wv, R, T, selg)


def _make_scatter_body(nslabs):
    def body_fn(*refs):
        slab_refs = [tuple(refs[3 * t:3 * t + 3]) for t in range(nslabs)]
        zer_hbm, am_hbm, az_hbm = refs[3 * nslabs:3 * nslabs + 3]
        (didxa, didx8a, didxb, didx8b, mra, zra, mrb, zrb,
         accm, accz, lma, lza, lmb, lzb) = refs[3 * nslabs + 3:]
        c = lax.axis_index("c")
        s = lax.axis_index("s")
        w = s * NC + c
        npr = BASE_PAIRS_S + jnp.where(w < EXTRA_W_S, 1, 0)

        pltpu.sync_copy(zer_hbm,
                        accm.at[pl.ds(s * ROWS_PER_TILE, ROWS_PER_TILE)])

        @pl.when(s < NAZ // 8)
        def _():
            pltpu.sync_copy(zer_hbm.at[pl.ds(0, 8)], accz.at[pl.ds(s * 8, 8)])

        plsc.subcore_barrier()

        for m_hbm, ze_hbm, dst_hbm in slab_refs:
            def body(i, carry):
                base = (w + i * NW) * 2 * CHS
                cma = pltpu.async_copy(m_hbm.at[pl.ds(base, CHS)], mra, lma)
                cza = pltpu.async_copy(ze_hbm.at[pl.ds(base, CHS)], zra, lza)
                cmb = pltpu.async_copy(
                    m_hbm.at[pl.ds(base + CHS, CHS)], mrb, lmb)
                czb = pltpu.async_copy(
                    ze_hbm.at[pl.ds(base + CHS, CHS)], zrb, lzb)
                pltpu.sync_copy(dst_hbm.at[pl.ds(base, CHS)], didxa)
                pltpu.sync_copy(dst_hbm.at[pl.ds(base + CHS, CHS)], didxb)
                for j in range(CHS // 16):
                    didx8a[pl.ds(j * 16, 16)] = lax.shift_right_logical(
                        didxa[pl.ds(j * 16, 16)], 7)
                    didx8b[pl.ds(j * 16, 16)] = lax.shift_right_logical(
                        didxb[pl.ds(j * 16, 16)], 7)
                cma.wait()
                cza.wait()
                pltpu.sync_copy(mra, accm.at[didxa], add=True)
                pltpu.sync_copy(zra, accz.at[didx8a], add=True)
                cmb.wait()
                czb.wait()
                pltpu.sync_copy(mrb, accm.at[didxb], add=True)
                pltpu.sync_copy(zrb, accz.at[didx8b], add=True)
                return carry

            lax.fori_loop(0, npr, body, 0)

        plsc.subcore_barrier()
        pltpu.sync_copy(
            accm.at[pl.ds(s * ROWS_PER_TILE, ROWS_PER_TILE)],
            am_hbm.at[c, pl.ds(s * ROWS_PER_TILE, ROWS_PER_TILE)])

        @pl.when(s < NAZ // 8)
        def _():
            pltpu.sync_copy(accz.at[pl.ds(s * 8, 8)],
                            az_hbm.at[c, pl.ds(s * 8, 8)])

    return body_fn


def _run_scatter(slabs, zer):
    fn = pl.kernel(
        _make_scatter_body(len(slabs)),
        out_type=(
            jax.ShapeDtypeStruct((NC, NP, D), jnp.float32),
            jax.ShapeDtypeStruct((NC, NAZ, D), jnp.float32),
        ),
        mesh=_mesh(),
        scratch_types=[
            pltpu.VMEM((CHS,), jnp.int32),
            pltpu.VMEM((CHS,), jnp.int32),
            pltpu.VMEM((CHS,), jnp.int32),
            pltpu.VMEM((CHS,), jnp.int32),
            pltpu.VMEM((CHS, D), jnp.float32),
            pltpu.VMEM((CHS, D), jnp.float32),
            pltpu.VMEM((CHS, D), jnp.float32),
            pltpu.VMEM((CHS, D), jnp.float32),
            pltpu.VMEM_SHARED((NP, D), jnp.float32),
            pltpu.VMEM_SHARED((NAZ, D), jnp.float32),
        ] + [pltpu.SemaphoreType.DMA] * 4,
    )
    args = []
    for t in slabs:
        args.extend(t)
    return fn(*args, zer)


def _body_c(am1_ref, az1_ref, am2_ref, az2_ref, sc_ref, rsel_ref, out_ref):
    stot = am1_ref[0] + am1_ref[1] + am2_ref[0] + am2_ref[1]
    azs = az1_ref[0] + az1_ref[1] + az2_ref[0] + az2_ref[1]
    b1 = jnp.dot(rsel_ref[...], azs)
    rowi = lax.broadcasted_iota(jnp.int32, (BNC, 1), 0)
    lane = lax.broadcasted_iota(jnp.int32, (BNC, D), 1)
    zmask = lane == (rowi & 127)
    z = jnp.sum(jnp.where(zmask, b1, 0.0), axis=1, keepdims=True)
    z = jnp.where(z == 0.0, 1.0, z)
    out_ref[...] = stot * lax.rsqrt(z) + sc_ref[...]


def _run_c(am1, az1, am2, az2, sc_pad, rsel):
    return pl.pallas_call(
        _body_c,
        grid=(NP // BNC,),
        in_specs=[
            pl.BlockSpec((NC, BNC, D), lambda i: (0, i, 0)),
            pl.BlockSpec((NC, BZ, D), lambda i: (0, i, 0)),
            pl.BlockSpec((NC, BNC, D), lambda i: (0, i, 0)),
            pl.BlockSpec((NC, BZ, D), lambda i: (0, i, 0)),
            pl.BlockSpec((BNC, D), lambda i: (i, 0)),
            pl.BlockSpec((BNC, BZ), lambda i: (0, 0)),
        ],
        out_specs=pl.BlockSpec((BNC, D), lambda i: (i, 0)),
        out_shape=jax.ShapeDtypeStruct((NP, D), jnp.float32),
    )(am1, az1, am2, az2, sc_pad, rsel)


_R_EXPAND = np.repeat(np.eye(H, dtype=np.float32), DE, axis=1)
_T_EXPAND = np.tile(np.eye(DE, dtype=np.float32), (1, H))
_SELG = np.zeros((D, 1), dtype=np.float32)
_SELG[GCOL, 0] = 1.0
_RSEL = np.zeros((BNC, BZ), dtype=np.float32)
_RSEL[np.arange(BNC), np.arange(BNC) // 128] = 1.0


@jax.jit
def kernel(node_features, node_attrs, edge_embedding, edge_attrs, edge_index,
           positions, Wq, W1k, W2k, Wk, W1v, W2v, Wv, Wdot, Wsc):
    src = edge_index[0].astype(jnp.int32)
    dst = edge_index[1].astype(jnp.int32)

    Wsc_t = jnp.transpose(Wsc, (1, 0, 2))
    W2k_r = jnp.transpose(W2k, (0, 2, 1)).reshape(H * DE, D)
    W2v_r = jnp.transpose(W2v, (0, 2, 1)).reshape(H * DE, D)
    Wdot_pad = jnp.concatenate(
        [Wdot, jnp.zeros((DQK, D - DQK), jnp.float32)], axis=1)
    Wk_pad = jnp.concatenate(
        [Wk, jnp.zeros((D, D - DQK), jnp.float32)], axis=1)
    R = jnp.asarray(_R_EXPAND)
    T = jnp.asarray(_T_EXPAND)
    selg = jnp.asarray(_SELG)
    rsel = jnp.asarray(_RSEL)
    zer = jnp.zeros((ROWS_PER_TILE, D), jnp.float32)

    qd = _run_a1(node_features, Wq, Wdot_pad)
    sc = _run_a2(node_features, node_attrs, Wsc_t)

    ms, zes, ds = [], [], []
    for sl in range(NSLAB):
        lo, hi = sl * SLAB, (sl + 1) * SLAB
        d_s = dst[lo:hi]
        xs, qdd = _run_gather(node_features, qd, src[lo:hi], d_s)
        m, ze = _run_b(xs, qdd, edge_embedding, edge_attrs,
                       W1k, W2k_r, Wk_pad, W1v, W2v_r, Wv, R, T, selg,
                       sl * (SLAB // BE))
        ms.append(m)
        zes.append(ze)
        ds.append(d_s)

    am1, az1 = _run_scatter(
        [(ms[0], zes[0], ds[0]), (ms[1], zes[1], ds[1]),
         (ms[2], zes[2], ds[2])], zer)
    am2, az2 = _run_scatter(
        [(ms[3], zes[3], ds[3]), (ms[4], zes[4], ds[4])], zer)
    sc_pad = jnp.pad(sc, ((0, NP - N), (0, 0)))
    return _run_c(am1, az1, am2, az2, sc_pad, rsel)[:N]
